# baseline (speedup 1.0000x reference)
import functools

import jax
import jax.numpy as jnp
from jax import lax
from jax.experimental import pallas as pl
from jax.experimental.pallas import tpu as pltpu
from jax.experimental.pallas import tpu_sc as plsc

N = 10000
E = 320000
D_IN = 128
HID = 256
D_OUT = 128

NC = 2
NS = 16
LANES = 16

CHUNK = 128
NRING = 3
DUMMY = N
N_PAD = 10240
RPS = N_PAD // NS
N_ACC = 10112
RPS_A = N_ACC // NS
E_PAD = 331776
CH_FULL = E_PAD // (NS * CHUNK)
CH_HALF = E_PAD // (NC * NS * CHUNK)
EROWS = E_PAD // CHUNK
CH_DEG = 88
EROWS_DEG = NC * NS * CH_DEG

_sc_mesh = plsc.VectorSubcoreMesh(
    core_axis_name="c", subcore_axis_name="s", num_cores=NC, num_subcores=NS)


@functools.partial(
    pl.kernel,
    out_type=jax.ShapeDtypeStruct((NC * N_PAD,), jnp.float32),
    mesh=_sc_mesh,
    compiler_params=pltpu.CompilerParams(needs_layout_passes=False),
    scratch_types=[
        pltpu.VMEM((CH_DEG, CHUNK), jnp.int32),
        pltpu.VMEM((N_PAD,), jnp.float32),
        pltpu.VMEM((NS, RPS), jnp.float32),
        pltpu.VMEM_SHARED((NS, N_PAD), jnp.float32),
    ],
)
def _deg_kernel(dst_hbm, cnt_hbm, didx, cnt_v, red_v, stage_s):
    c = lax.axis_index("c")
    s = lax.axis_index("s")
    zero16 = jnp.zeros((LANES,), jnp.float32)
    one16 = jnp.ones((LANES,), jnp.float32)

    row0 = (c * NS + s) * CH_DEG
    pltpu.sync_copy(dst_hbm.at[pl.ds(row0, CH_DEG)], didx)

    @pl.loop(0, N_PAD // LANES)
    def _zero(i):
        cnt_v[pl.ds(i * LANES, LANES)] = zero16

    @pl.loop(0, CH_DEG)
    def _count(i):
        for j in range(CHUNK // LANES):
            idx = didx[i, pl.ds(j * LANES, LANES)]
            plsc.addupdate_scatter(cnt_v, [idx], one16)

    pltpu.sync_copy(cnt_v, stage_s.at[s])
    plsc.subcore_barrier()

    rbase = s * RPS
    pltpu.sync_copy(stage_s.at[:, pl.ds(rbase, RPS)], red_v)

    @pl.loop(0, RPS // LANES)
    def _reduce(i):
        sl = pl.ds(i * LANES, LANES)
        acc16 = red_v[0, sl]
        for t in range(1, NS):
            acc16 = acc16 + red_v[t, sl]
        cnt_v[sl] = acc16

    pltpu.sync_copy(cnt_v.at[pl.ds(0, RPS)],
                    cnt_hbm.at[pl.ds(c * N_PAD + rbase, RPS)])


def _edge_ring(g_hbm, src2d, dst2d, rs0, rd0, acc,
               sidx, didx, rows, sem_is, sem_id, sem_g, sem_s, n_chunks):
    def wait_s(t):
        pltpu.make_async_copy(rows[t], acc.at[pl.ds(0, CHUNK)], sem_s.at[t]).wait()

    def wait_g(t):
        pltpu.make_async_copy(g_hbm.at[pl.ds(0, CHUNK)], rows[t], sem_g.at[t]).wait()

    def wait_is(t):
        pltpu.make_async_copy(src2d.at[rs0], sidx.at[t], sem_is.at[t]).wait()

    def wait_id(t):
        pltpu.make_async_copy(dst2d.at[rd0], didx.at[t], sem_id.at[t]).wait()

    for q in range(NRING):
        pltpu.async_copy(src2d.at[rs0 + q], sidx.at[q], sem_is.at[q])
    for q in range(NRING - 1):
        pltpu.async_copy(dst2d.at[rd0 + q], didx.at[q], sem_id.at[q])
    for q in range(2):
        wait_is(q)
        pltpu.async_copy(g_hbm.at[sidx.at[q]], rows[q], sem_g.at[q])

    @pl.loop(0, n_chunks // NRING)
    def _grp(grp):
        for t in range(NRING):
            i = grp * NRING + t
            tm1 = (t + NRING - 1) % NRING

            @pl.when(i > 0)
            def _():
                wait_s(tm1)

            @pl.when(i + 2 < n_chunks)
            def _():
                pltpu.async_copy(dst2d.at[rd0 + i + 2], didx.at[tm1],
                                 sem_id.at[tm1])

            wait_g(t)

            @pl.when(i + NRING < n_chunks)
            def _():
                pltpu.async_copy(src2d.at[rs0 + i + NRING], sidx.at[t],
                                 sem_is.at[t])

            wait_id(t)
            pltpu.async_copy(rows[t], acc.at[didx.at[t]], sem_s.at[t], add=True)

            @pl.when(i + 2 < n_chunks)
            def _():
                wait_is(tm1)
                pltpu.async_copy(g_hbm.at[sidx.at[tm1]], rows[tm1],
                                 sem_g.at[tm1])

    wait_s((n_chunks - 1) % NRING)


@functools.partial(
    pl.kernel,
    out_type=jax.ShapeDtypeStruct((NC * N_PAD, 128), jnp.float32),
    mesh=_sc_mesh,
    scratch_types=[
        pltpu.VMEM((NRING, CHUNK), jnp.int32),
        pltpu.VMEM((NRING, CHUNK), jnp.int32),
        [pltpu.VMEM((CHUNK, 128), jnp.float32)] * NRING,
        pltpu.VMEM_SHARED((N_ACC, 128), jnp.float32),
        pltpu.SemaphoreType.DMA((NRING,)),
        pltpu.SemaphoreType.DMA((NRING,)),
        pltpu.SemaphoreType.DMA((NRING,)),
        pltpu.SemaphoreType.DMA((NRING,)),
    ],
)
def _prop_wide_kernel(g_hbm, srcb_hbm, dst_hbm, out_hbm,
                      sidx, didx, rows, acc, sem_is, sem_id, sem_g, sem_s):
    c = lax.axis_index("c")
    s = lax.axis_index("s")
    rbase = s * RPS_A
    pltpu.sync_copy(g_hbm.at[pl.ds(c * N_PAD + rbase, RPS_A)],
                    acc.at[pl.ds(rbase, RPS_A)])
    plsc.subcore_barrier()

    _edge_ring(g_hbm, srcb_hbm, dst_hbm, c * EROWS + s * CH_FULL, s * CH_FULL,
               acc, sidx, didx, rows, sem_is, sem_id, sem_g, sem_s, CH_FULL)

    plsc.subcore_barrier()
    pltpu.sync_copy(acc.at[pl.ds(rbase, RPS_A)],
                    out_hbm.at[pl.ds(c * N_PAD + rbase, RPS_A)])


@functools.partial(
    pl.kernel,
    out_type=jax.ShapeDtypeStruct((NC * N_PAD, 128), jnp.float32),
    mesh=_sc_mesh,
    scratch_types=[
        pltpu.VMEM((NRING, CHUNK), jnp.int32),
        pltpu.VMEM((NRING, CHUNK), jnp.int32),
        [pltpu.VMEM((CHUNK, 128), jnp.float32)] * NRING,
        pltpu.VMEM_SHARED((N_ACC, 128), jnp.float32),
        pltpu.SemaphoreType.DMA((NRING,)),
        pltpu.SemaphoreType.DMA((NRING,)),
        pltpu.SemaphoreType.DMA((NRING,)),
        pltpu.SemaphoreType.DMA((NRING,)),
    ],
)
def _prop_narrow_kernel(g_hbm, src_hbm, dst_hbm, out_hbm,
                        sidx, didx, rows, acc, sem_is, sem_id, sem_g, sem_s):
    c = lax.axis_index("c")
    s = lax.axis_index("s")
    rbase = s * RPS_A
    row0 = (c * NS + s) * CH_HALF
    pltpu.sync_copy(g_hbm.at[pl.ds(rbase, RPS_A)], acc.at[pl.ds(rbase, RPS_A)])
    plsc.subcore_barrier()

    _edge_ring(g_hbm, src_hbm, dst_hbm, row0, row0, acc,
               sidx, didx, rows, sem_is, sem_id, sem_g, sem_s, CH_HALF)

    plsc.subcore_barrier()
    pltpu.sync_copy(acc.at[pl.ds(rbase, RPS_A)],
                    out_hbm.at[pl.ds(c * N_PAD + rbase, RPS_A)])


R = 1000


def _tc_in_body(x_ref, w_ref, cnt0_ref, cnt1_ref, g_ref, dis_ref):
    deg = 1.0 + cnt0_ref[0] + cnt1_ref[0]
    dis = lax.rsqrt(deg)
    dis_ref[...] = dis
    h = jnp.dot(x_ref[...], w_ref[...], preferred_element_type=jnp.float32)
    g_ref[0] = h * dis


def _tc_layer1(x, W1, cnt):
    return pl.pallas_call(
        _tc_in_body,
        grid=(N // R, NC),
        in_specs=[
            pl.BlockSpec((R, D_IN), lambda r, c: (r, 0)),
            pl.BlockSpec((D_IN, 128), lambda r, c: (0, c)),
            pl.BlockSpec((1, R, 1), lambda r, c: (0, r, 0)),
            pl.BlockSpec((1, R, 1), lambda r, c: (1, r, 0)),
        ],
        out_specs=[
            pl.BlockSpec((1, R, 128), lambda r, c: (c, r, 0)),
            pl.BlockSpec((R, 1), lambda r, c: (r, 0)),
        ],
        out_shape=[
            jax.ShapeDtypeStruct((NC, N_PAD, 128), jnp.float32),
            jax.ShapeDtypeStruct((N, 1), jnp.float32),
        ],
    )(x, W1, cnt, cnt)


def _tc_mid_body(a0_ref, a1_ref, dis_ref, b_ref, w_ref, h_ref, g_ref):
    dis = dis_ref[...]
    h = jnp.concatenate([a0_ref[0], a1_ref[0]], axis=1)
    h = jnp.maximum(h * dis + b_ref[...], 0.0)
    h_ref[...] = h
    g = jnp.dot(h, w_ref[...], preferred_element_type=jnp.float32)
    g_ref[0] = g * dis


def _tc_mid(a, dis, b, W):
    return pl.pallas_call(
        _tc_mid_body,
        grid=(N // R, NC),
        in_specs=[
            pl.BlockSpec((1, R, 128), lambda r, c: (0, r, 0)),
            pl.BlockSpec((1, R, 128), lambda r, c: (1, r, 0)),
            pl.BlockSpec((R, 1), lambda r, c: (r, 0)),
            pl.BlockSpec((1, HID), lambda r, c: (0, 0)),
            pl.BlockSpec((HID, 128), lambda r, c: (0, c)),
        ],
        out_specs=[
            pl.BlockSpec((R, HID), lambda r, c: (r, 0)),
            pl.BlockSpec((1, R, 128), lambda r, c: (c, r, 0)),
        ],
        out_shape=[
            jax.ShapeDtypeStruct((N, HID), jnp.float32),
            jax.ShapeDtypeStruct((NC, N_PAD, 128), jnp.float32),
        ],
    )(a, a, dis, b.reshape(1, HID), W)


def _tc_last_body(a0_ref, a1_ref, dis_ref, b_ref, w_ref, h_ref, g_ref):
    dis = dis_ref[...]
    h = jnp.concatenate([a0_ref[0], a1_ref[0]], axis=1)
    h = jnp.maximum(h * dis + b_ref[...], 0.0)
    h_ref[...] = h
    g = jnp.dot(h, w_ref[...], preferred_element_type=jnp.float32)
    g_ref[...] = g * dis


def _tc_last(a, dis, b, W):
    return pl.pallas_call(
        _tc_last_body,
        grid=(N // R,),
        in_specs=[
            pl.BlockSpec((1, R, 128), lambda r: (0, r, 0)),
            pl.BlockSpec((1, R, 128), lambda r: (1, r, 0)),
            pl.BlockSpec((R, 1), lambda r: (r, 0)),
            pl.BlockSpec((1, HID), lambda r: (0, 0)),
            pl.BlockSpec((HID, D_OUT), lambda r: (0, 0)),
        ],
        out_specs=[
            pl.BlockSpec((R, HID), lambda r: (r, 0)),
            pl.BlockSpec((R, D_OUT), lambda r: (r, 0)),
        ],
        out_shape=[
            jax.ShapeDtypeStruct((N, HID), jnp.float32),
            jax.ShapeDtypeStruct((N_PAD, D_OUT), jnp.float32),
        ],
    )(a, a, dis, b.reshape(1, HID), W)


def _tc_final_body(a0_ref, a1_ref, g_ref, dis_ref, b_ref, o_ref):
    o_ref[...] = (a0_ref[0] + a1_ref[0] - g_ref[...]) * dis_ref[...] + b_ref[...]


def _tc_final(a, g3, dis, b):
    return pl.pallas_call(
        _tc_final_body,
        grid=(N // R,),
        in_specs=[
            pl.BlockSpec((1, R, D_OUT), lambda r: (0, r, 0)),
            pl.BlockSpec((1, R, D_OUT), lambda r: (1, r, 0)),
            pl.BlockSpec((R, D_OUT), lambda r: (r, 0)),
            pl.BlockSpec((R, 1), lambda r: (r, 0)),
            pl.BlockSpec((1, D_OUT), lambda r: (0, 0)),
        ],
        out_specs=pl.BlockSpec((R, D_OUT), lambda r: (r, 0)),
        out_shape=jax.ShapeDtypeStruct((N, D_OUT), jnp.float32),
    )(a, a, g3, dis, b.reshape(1, D_OUT))


@jax.jit
def kernel(x, edge_index, W1, b1, W2, b2, W3, b3):
    npad = E_PAD - E
    src_p = jnp.concatenate([edge_index[0], jnp.zeros((npad,), jnp.int32)])
    dst_p = jnp.concatenate([edge_index[1], jnp.full((npad,), DUMMY, jnp.int32)])
    src_both = jnp.concatenate([src_p, src_p + N_PAD]).reshape(2 * EROWS, CHUNK)
    src2 = src_p.reshape(EROWS, CHUNK)
    dst2 = dst_p.reshape(EROWS, CHUNK)
    dst_deg = jnp.concatenate(
        [dst_p, jnp.full((EROWS_DEG * CHUNK - E_PAD,), DUMMY, jnp.int32)]
    ).reshape(EROWS_DEG, CHUNK)

    cnt = _deg_kernel(dst_deg).reshape(NC, N_PAD, 1)
    g1, dis = _tc_layer1(x, W1, cnt)

    a1 = _prop_wide_kernel(g1.reshape(NC * N_PAD, 128), src_both, dst2)
    h1, g2 = _tc_mid(a1.reshape(NC, N_PAD, 128), dis, b1, W2)

    a2 = _prop_wide_kernel(g2.reshape(NC * N_PAD, 128), src_both, dst2)
    h2, g3 = _tc_last(a2.reshape(NC, N_PAD, 128), dis, b2, W3)

    a3 = _prop_narrow_kernel(g3, src2, dst2)
    out = _tc_final(a3.reshape(NC, N_PAD, 128), g3, dis, b3)
    return (out, h1, h2)


# speedup vs baseline: 1.5202x; 1.5202x over previous
import functools

import jax
import jax.numpy as jnp
from jax import lax
from jax.experimental import pallas as pl
from jax.experimental.pallas import tpu as pltpu
from jax.experimental.pallas import tpu_sc as plsc

N = 10000
E = 320000
D_IN = 128
HID = 256
D_OUT = 128

NC = 2
NS = 16
LANES = 16

CHUNK = 128
NBUF = 2
DUMMY = N
N_PAD = 10240
RPS = N_PAD // NS
E_PAD = 327680
CH_FULL = E_PAD // (NS * CHUNK)
CH_HALF = E_PAD // (NC * NS * CHUNK)
EROWS = E_PAD // CHUNK

_sc_mesh = plsc.VectorSubcoreMesh(
    core_axis_name="c", subcore_axis_name="s", num_cores=NC, num_subcores=NS)


@functools.partial(
    pl.kernel,
    out_type=jax.ShapeDtypeStruct((NC * N_PAD,), jnp.float32),
    mesh=_sc_mesh,
    compiler_params=pltpu.CompilerParams(needs_layout_passes=False),
    scratch_types=[
        pltpu.VMEM((CH_HALF, CHUNK), jnp.int32),
        pltpu.VMEM((N_PAD,), jnp.float32),
        pltpu.VMEM((NS, RPS), jnp.float32),
        pltpu.VMEM_SHARED((NS, N_PAD), jnp.float32),
    ],
)
def _deg_kernel(dst_hbm, cnt_hbm, didx, cnt_v, red_v, stage_s):
    c = lax.axis_index("c")
    s = lax.axis_index("s")
    zero16 = jnp.zeros((LANES,), jnp.float32)
    one16 = jnp.ones((LANES,), jnp.float32)

    row0 = (c * NS + s) * CH_HALF
    pltpu.sync_copy(dst_hbm.at[pl.ds(row0, CH_HALF)], didx)

    @pl.loop(0, N_PAD // LANES)
    def _zero(i):
        cnt_v[pl.ds(i * LANES, LANES)] = zero16

    @pl.loop(0, CH_HALF)
    def _count(i):
        for j in range(CHUNK // LANES):
            idx = didx[i, pl.ds(j * LANES, LANES)]
            plsc.addupdate_scatter(cnt_v, [idx], one16)

    pltpu.sync_copy(cnt_v, stage_s.at[s])
    plsc.subcore_barrier()

    rbase = s * RPS
    pltpu.sync_copy(stage_s.at[:, pl.ds(rbase, RPS)], red_v)

    @pl.loop(0, RPS // LANES)
    def _reduce(i):
        sl = pl.ds(i * LANES, LANES)
        acc16 = red_v[0, sl]
        for t in range(1, NS):
            acc16 = acc16 + red_v[t, sl]
        cnt_v[sl] = acc16

    pltpu.sync_copy(cnt_v.at[pl.ds(0, RPS)],
                    cnt_hbm.at[pl.ds(c * N_PAD + rbase, RPS)])


NIDX = 4


def _edge_ring(g_hbm, src2d, dst2d, rs0, rd0, acc,
               sidx, didx, rows, sem_i, sem_g, sem_s, n_chunks):
    for q in range(NIDX - 1):
        pltpu.async_copy(src2d.at[rs0 + q], sidx.at[q], sem_i.at[q])
        pltpu.async_copy(dst2d.at[rd0 + q], didx.at[q], sem_i.at[q])
    pltpu.make_async_copy(src2d.at[rs0], sidx.at[0], sem_i.at[0]).wait()
    pltpu.make_async_copy(dst2d.at[rd0], didx.at[0], sem_i.at[0]).wait()
    pltpu.async_copy(g_hbm.at[sidx.at[0]], rows[0], sem_g.at[0])

    @pl.loop(0, n_chunks // NIDX)
    def _grp(grp):
        for q in range(NIDX):
            i = grp * NIDX + q
            b = q % NBUF
            o = 1 - b
            qn = (q + 1) % NIDX

            @pl.when(i > 0)
            def _():
                pltpu.make_async_copy(
                    rows[o], acc.at[pl.ds(0, CHUNK)], sem_s.at[o]).wait()

            @pl.when(i + NIDX - 1 < n_chunks)
            def _():
                qp = (q + NIDX - 1) % NIDX
                pltpu.async_copy(src2d.at[rs0 + i + NIDX - 1],
                                 sidx.at[qp], sem_i.at[qp])
                pltpu.async_copy(dst2d.at[rd0 + i + NIDX - 1],
                                 didx.at[qp], sem_i.at[qp])

            pltpu.make_async_copy(
                g_hbm.at[pl.ds(0, CHUNK)], rows[b], sem_g.at[b]).wait()
            pltpu.async_copy(rows[b], acc.at[didx.at[q]], sem_s.at[b], add=True)

            @pl.when(i + 1 < n_chunks)
            def _():
                pltpu.make_async_copy(
                    src2d.at[rs0], sidx.at[qn], sem_i.at[qn]).wait()
                pltpu.make_async_copy(
                    dst2d.at[rd0], didx.at[qn], sem_i.at[qn]).wait()
                pltpu.async_copy(g_hbm.at[sidx.at[qn]], rows[o], sem_g.at[o])

    pltpu.make_async_copy(
        rows[(n_chunks - 1) % NBUF], acc.at[pl.ds(0, CHUNK)],
        sem_s.at[(n_chunks - 1) % NBUF]).wait()


@functools.partial(
    pl.kernel,
    out_type=jax.ShapeDtypeStruct((NC * N_PAD, 128), jnp.float32),
    mesh=_sc_mesh,
    scratch_types=[
        pltpu.VMEM((NIDX, CHUNK), jnp.int32),
        pltpu.VMEM((NIDX, CHUNK), jnp.int32),
        [pltpu.VMEM((CHUNK, 128), jnp.float32)] * NBUF,
        pltpu.VMEM_SHARED((N_PAD, 128), jnp.float32),
        pltpu.SemaphoreType.DMA((NIDX,)),
        pltpu.SemaphoreType.DMA((NBUF,)),
        pltpu.SemaphoreType.DMA((NBUF,)),
    ],
)
def _prop_wide_kernel(g_hbm, srcb_hbm, dst_hbm, out_hbm,
                      sidx, didx, rows, acc, sem_i, sem_g, sem_s):
    c = lax.axis_index("c")
    s = lax.axis_index("s")
    rbase = s * RPS
    pltpu.sync_copy(g_hbm.at[pl.ds(c * N_PAD + rbase, RPS)],
                    acc.at[pl.ds(rbase, RPS)])
    plsc.subcore_barrier()

    _edge_ring(g_hbm, srcb_hbm, dst_hbm, c * EROWS + s * CH_FULL, s * CH_FULL,
               acc, sidx, didx, rows, sem_i, sem_g, sem_s, CH_FULL)

    plsc.subcore_barrier()
    pltpu.sync_copy(acc.at[pl.ds(rbase, RPS)],
                    out_hbm.at[pl.ds(c * N_PAD + rbase, RPS)])


@functools.partial(
    pl.kernel,
    out_type=jax.ShapeDtypeStruct((NC * N_PAD, 128), jnp.float32),
    mesh=_sc_mesh,
    scratch_types=[
        pltpu.VMEM((NIDX, CHUNK), jnp.int32),
        pltpu.VMEM((NIDX, CHUNK), jnp.int32),
        [pltpu.VMEM((CHUNK, 128), jnp.float32)] * NBUF,
        pltpu.VMEM_SHARED((N_PAD, 128), jnp.float32),
        pltpu.SemaphoreType.DMA((NIDX,)),
        pltpu.SemaphoreType.DMA((NBUF,)),
        pltpu.SemaphoreType.DMA((NBUF,)),
    ],
)
def _prop_narrow_kernel(g_hbm, src_hbm, dst_hbm, out_hbm,
                        sidx, didx, rows, acc, sem_i, sem_g, sem_s):
    c = lax.axis_index("c")
    s = lax.axis_index("s")
    rbase = s * RPS
    row0 = (c * NS + s) * CH_HALF
    pltpu.sync_copy(g_hbm.at[pl.ds(rbase, RPS)], acc.at[pl.ds(rbase, RPS)])
    plsc.subcore_barrier()

    _edge_ring(g_hbm, src_hbm, dst_hbm, row0, row0, acc,
               sidx, didx, rows, sem_i, sem_g, sem_s, CH_HALF)

    plsc.subcore_barrier()
    pltpu.sync_copy(acc.at[pl.ds(rbase, RPS)],
                    out_hbm.at[pl.ds(c * N_PAD + rbase, RPS)])


R = 1000


def _tc_in_body(x_ref, w_ref, cnt0_ref, cnt1_ref, g_ref, dis_ref):
    deg = 1.0 + cnt0_ref[0] + cnt1_ref[0]
    dis = lax.rsqrt(deg)
    dis_ref[...] = dis
    h = jnp.dot(x_ref[...], w_ref[...], preferred_element_type=jnp.float32)
    g_ref[0] = h * dis


def _tc_layer1(x, W1, cnt):
    return pl.pallas_call(
        _tc_in_body,
        grid=(N // R, NC),
        in_specs=[
            pl.BlockSpec((R, D_IN), lambda r, c: (r, 0)),
            pl.BlockSpec((D_IN, 128), lambda r, c: (0, c)),
            pl.BlockSpec((1, R, 1), lambda r, c: (0, r, 0)),
            pl.BlockSpec((1, R, 1), lambda r, c: (1, r, 0)),
        ],
        out_specs=[
            pl.BlockSpec((1, R, 128), lambda r, c: (c, r, 0)),
            pl.BlockSpec((R, 1), lambda r, c: (r, 0)),
        ],
        out_shape=[
            jax.ShapeDtypeStruct((NC, N_PAD, 128), jnp.float32),
            jax.ShapeDtypeStruct((N, 1), jnp.float32),
        ],
    )(x, W1, cnt, cnt)


def _tc_mid_body(a0_ref, a1_ref, dis_ref, b_ref, w_ref, h_ref, g_ref):
    dis = dis_ref[...]
    h = jnp.concatenate([a0_ref[0], a1_ref[0]], axis=1)
    h = jnp.maximum(h * dis + b_ref[...], 0.0)
    h_ref[...] = h
    g = jnp.dot(h, w_ref[...], preferred_element_type=jnp.float32)
    g_ref[0] = g * dis


def _tc_mid(a, dis, b, W):
    return pl.pallas_call(
        _tc_mid_body,
        grid=(N // R, NC),
        in_specs=[
            pl.BlockSpec((1, R, 128), lambda r, c: (0, r, 0)),
            pl.BlockSpec((1, R, 128), lambda r, c: (1, r, 0)),
            pl.BlockSpec((R, 1), lambda r, c: (r, 0)),
            pl.BlockSpec((1, HID), lambda r, c: (0, 0)),
            pl.BlockSpec((HID, 128), lambda r, c: (0, c)),
        ],
        out_specs=[
            pl.BlockSpec((R, HID), lambda r, c: (r, 0)),
            pl.BlockSpec((1, R, 128), lambda r, c: (c, r, 0)),
        ],
        out_shape=[
            jax.ShapeDtypeStruct((N, HID), jnp.float32),
            jax.ShapeDtypeStruct((NC, N_PAD, 128), jnp.float32),
        ],
    )(a, a, dis, b.reshape(1, HID), W)


def _tc_last_body(a0_ref, a1_ref, dis_ref, b_ref, w_ref, h_ref, g_ref):
    dis = dis_ref[...]
    h = jnp.concatenate([a0_ref[0], a1_ref[0]], axis=1)
    h = jnp.maximum(h * dis + b_ref[...], 0.0)
    h_ref[...] = h
    g = jnp.dot(h, w_ref[...], preferred_element_type=jnp.float32)
    g_ref[...] = g * dis


def _tc_last(a, dis, b, W):
    return pl.pallas_call(
        _tc_last_body,
        grid=(N // R,),
        in_specs=[
            pl.BlockSpec((1, R, 128), lambda r: (0, r, 0)),
            pl.BlockSpec((1, R, 128), lambda r: (1, r, 0)),
            pl.BlockSpec((R, 1), lambda r: (r, 0)),
            pl.BlockSpec((1, HID), lambda r: (0, 0)),
            pl.BlockSpec((HID, D_OUT), lambda r: (0, 0)),
        ],
        out_specs=[
            pl.BlockSpec((R, HID), lambda r: (r, 0)),
            pl.BlockSpec((R, D_OUT), lambda r: (r, 0)),
        ],
        out_shape=[
            jax.ShapeDtypeStruct((N, HID), jnp.float32),
            jax.ShapeDtypeStruct((N_PAD, D_OUT), jnp.float32),
        ],
    )(a, a, dis, b.reshape(1, HID), W)


def _tc_final_body(a0_ref, a1_ref, g_ref, dis_ref, b_ref, o_ref):
    o_ref[...] = (a0_ref[0] + a1_ref[0] - g_ref[...]) * dis_ref[...] + b_ref[...]


def _tc_final(a, g3, dis, b):
    return pl.pallas_call(
        _tc_final_body,
        grid=(N // R,),
        in_specs=[
            pl.BlockSpec((1, R, D_OUT), lambda r: (0, r, 0)),
            pl.BlockSpec((1, R, D_OUT), lambda r: (1, r, 0)),
            pl.BlockSpec((R, D_OUT), lambda r: (r, 0)),
            pl.BlockSpec((R, 1), lambda r: (r, 0)),
            pl.BlockSpec((1, D_OUT), lambda r: (0, 0)),
        ],
        out_specs=pl.BlockSpec((R, D_OUT), lambda r: (r, 0)),
        out_shape=jax.ShapeDtypeStruct((N, D_OUT), jnp.float32),
    )(a, a, g3, dis, b.reshape(1, D_OUT))


@jax.jit
def kernel(x, edge_index, W1, b1, W2, b2, W3, b3):
    npad = E_PAD - E
    src_p = jnp.concatenate([edge_index[0], jnp.zeros((npad,), jnp.int32)])
    pad_dst = DUMMY + jnp.arange(npad, dtype=jnp.int32) % (N_PAD - N)
    dst_p = jnp.concatenate([edge_index[1], pad_dst])
    src_both = jnp.concatenate([src_p, src_p + N_PAD]).reshape(2 * EROWS, CHUNK)
    src2 = src_p.reshape(EROWS, CHUNK)
    dst2 = dst_p.reshape(EROWS, CHUNK)

    cnt = _deg_kernel(dst2).reshape(NC, N_PAD, 1)
    g1, dis = _tc_layer1(x, W1, cnt)

    a1 = _prop_wide_kernel(g1.reshape(NC * N_PAD, 128), src_both, dst2)
    h1, g2 = _tc_mid(a1.reshape(NC, N_PAD, 128), dis, b1, W2)

    a2 = _prop_wide_kernel(g2.reshape(NC * N_PAD, 128), src_both, dst2)
    h2, g3 = _tc_last(a2.reshape(NC, N_PAD, 128), dis, b2, W3)

    a3 = _prop_narrow_kernel(g3, src2, dst2)
    out = _tc_final(a3.reshape(NC, N_PAD, 128), g3, dis, b3)
    return (out, h1, h2)


# speedup vs baseline: 1.6672x; 1.0967x over previous
import functools

import jax
import jax.numpy as jnp
from jax import lax
from jax.experimental import pallas as pl
from jax.experimental.pallas import tpu as pltpu
from jax.experimental.pallas import tpu_sc as plsc

N = 10000
E = 320000
D_IN = 128
HID = 256
D_OUT = 128

NC = 2
NS = 16
LANES = 16

CHUNK = 128
NBUF = 2
DUMMY = N
N_PAD = 10240
RPS = N_PAD // NS
E_PAD = 327680
CH_FULL = E_PAD // (NS * CHUNK)
CH_HALF = E_PAD // (NC * NS * CHUNK)
EROWS = E_PAD // CHUNK

_sc_mesh = plsc.VectorSubcoreMesh(
    core_axis_name="c", subcore_axis_name="s", num_cores=NC, num_subcores=NS)


@functools.partial(
    pl.kernel,
    out_type=jax.ShapeDtypeStruct((NC * N_PAD,), jnp.float32),
    mesh=_sc_mesh,
    compiler_params=pltpu.CompilerParams(needs_layout_passes=False),
    scratch_types=[
        pltpu.VMEM((CH_HALF, CHUNK), jnp.int32),
        pltpu.VMEM((N_PAD,), jnp.float32),
        pltpu.VMEM((NS, RPS), jnp.float32),
        pltpu.VMEM_SHARED((NS, N_PAD), jnp.float32),
    ],
)
def _deg_kernel(dst_hbm, cnt_hbm, didx, cnt_v, red_v, stage_s):
    c = lax.axis_index("c")
    s = lax.axis_index("s")
    zero16 = jnp.zeros((LANES,), jnp.float32)
    one16 = jnp.ones((LANES,), jnp.float32)

    row0 = (c * NS + s) * CH_HALF
    pltpu.sync_copy(dst_hbm.at[pl.ds(row0, CH_HALF)], didx)

    @pl.loop(0, N_PAD // LANES)
    def _zero(i):
        cnt_v[pl.ds(i * LANES, LANES)] = zero16

    @pl.loop(0, CH_HALF)
    def _count(i):
        for j in range(CHUNK // LANES):
            idx = didx[i, pl.ds(j * LANES, LANES)]
            plsc.addupdate_scatter(cnt_v, [idx], one16)

    pltpu.sync_copy(cnt_v, stage_s.at[s])
    plsc.subcore_barrier()

    rbase = s * RPS
    pltpu.sync_copy(stage_s.at[:, pl.ds(rbase, RPS)], red_v)

    @pl.loop(0, RPS // LANES)
    def _reduce(i):
        sl = pl.ds(i * LANES, LANES)
        acc16 = red_v[0, sl]
        for t in range(1, NS):
            acc16 = acc16 + red_v[t, sl]
        cnt_v[sl] = acc16

    pltpu.sync_copy(cnt_v.at[pl.ds(0, RPS)],
                    cnt_hbm.at[pl.ds(c * N_PAD + rbase, RPS)])


NIDX = 4


def _edge_ring(g_hbm, src2d, dst2d, rs0, rd0, stride, acc,
               sidx, didx, rows, sem_i, sem_g, sem_s, n_chunks):
    for q in range(NIDX - 1):
        pltpu.async_copy(src2d.at[rs0 + q * stride], sidx.at[q], sem_i.at[q])
        pltpu.async_copy(dst2d.at[rd0 + q * stride], didx.at[q], sem_i.at[q])
    pltpu.make_async_copy(src2d.at[rs0], sidx.at[0], sem_i.at[0]).wait()
    pltpu.make_async_copy(dst2d.at[rd0], didx.at[0], sem_i.at[0]).wait()
    pltpu.async_copy(g_hbm.at[sidx.at[0]], rows[0], sem_g.at[0])

    @pl.loop(0, n_chunks // NIDX)
    def _grp(grp):
        for q in range(NIDX):
            i = grp * NIDX + q
            b = q % NBUF
            o = 1 - b
            qn = (q + 1) % NIDX

            @pl.when(i > 0)
            def _():
                pltpu.make_async_copy(
                    rows[o], acc.at[pl.ds(0, CHUNK)], sem_s.at[o]).wait()

            @pl.when(i + NIDX - 1 < n_chunks)
            def _():
                qp = (q + NIDX - 1) % NIDX
                pltpu.async_copy(src2d.at[rs0 + (i + NIDX - 1) * stride],
                                 sidx.at[qp], sem_i.at[qp])
                pltpu.async_copy(dst2d.at[rd0 + (i + NIDX - 1) * stride],
                                 didx.at[qp], sem_i.at[qp])

            pltpu.make_async_copy(
                g_hbm.at[pl.ds(0, CHUNK)], rows[b], sem_g.at[b]).wait()
            pltpu.async_copy(rows[b], acc.at[didx.at[q]], sem_s.at[b], add=True)

            @pl.when(i + 1 < n_chunks)
            def _():
                pltpu.make_async_copy(
                    src2d.at[rs0], sidx.at[qn], sem_i.at[qn]).wait()
                pltpu.make_async_copy(
                    dst2d.at[rd0], didx.at[qn], sem_i.at[qn]).wait()
                pltpu.async_copy(g_hbm.at[sidx.at[qn]], rows[o], sem_g.at[o])

    pltpu.make_async_copy(
        rows[(n_chunks - 1) % NBUF], acc.at[pl.ds(0, CHUNK)],
        sem_s.at[(n_chunks - 1) % NBUF]).wait()


@functools.partial(
    pl.kernel,
    out_type=jax.ShapeDtypeStruct((NC * N_PAD, 128), jnp.float32),
    mesh=_sc_mesh,
    scratch_types=[
        pltpu.VMEM((NIDX, CHUNK), jnp.int32),
        pltpu.VMEM((NIDX, CHUNK), jnp.int32),
        [pltpu.VMEM((CHUNK, 128), jnp.float32)] * NBUF,
        pltpu.VMEM_SHARED((N_PAD, 128), jnp.float32),
        pltpu.SemaphoreType.DMA((NIDX,)),
        pltpu.SemaphoreType.DMA((NBUF,)),
        pltpu.SemaphoreType.DMA((NBUF,)),
    ],
)
def _prop_wide_kernel(g_hbm, srcb_hbm, dst_hbm, out_hbm,
                      sidx, didx, rows, acc, sem_i, sem_g, sem_s):
    c = lax.axis_index("c")
    s = lax.axis_index("s")
    rbase = s * RPS
    pltpu.sync_copy(g_hbm.at[pl.ds(c * N_PAD + rbase, RPS)],
                    acc.at[pl.ds(rbase, RPS)])
    plsc.subcore_barrier()

    _edge_ring(g_hbm, srcb_hbm, dst_hbm, c * EROWS + s, s, NS,
               acc, sidx, didx, rows, sem_i, sem_g, sem_s, CH_FULL)

    plsc.subcore_barrier()
    pltpu.sync_copy(acc.at[pl.ds(rbase, RPS)],
                    out_hbm.at[pl.ds(c * N_PAD + rbase, RPS)])


@functools.partial(
    pl.kernel,
    out_type=jax.ShapeDtypeStruct((NC * N_PAD, 128), jnp.float32),
    mesh=_sc_mesh,
    scratch_types=[
        pltpu.VMEM((NIDX, CHUNK), jnp.int32),
        pltpu.VMEM((NIDX, CHUNK), jnp.int32),
        [pltpu.VMEM((CHUNK, 128), jnp.float32)] * NBUF,
        pltpu.VMEM_SHARED((N_PAD, 128), jnp.float32),
        pltpu.SemaphoreType.DMA((NIDX,)),
        pltpu.SemaphoreType.DMA((NBUF,)),
        pltpu.SemaphoreType.DMA((NBUF,)),
    ],
)
def _prop_narrow_kernel(g_hbm, src_hbm, dst_hbm, out_hbm,
                        sidx, didx, rows, acc, sem_i, sem_g, sem_s):
    c = lax.axis_index("c")
    s = lax.axis_index("s")
    rbase = s * RPS
    w = c * NS + s
    pltpu.sync_copy(g_hbm.at[pl.ds(rbase, RPS)], acc.at[pl.ds(rbase, RPS)])
    plsc.subcore_barrier()

    _edge_ring(g_hbm, src_hbm, dst_hbm, w, w, NC * NS, acc,
               sidx, didx, rows, sem_i, sem_g, sem_s, CH_HALF)

    plsc.subcore_barrier()
    pltpu.sync_copy(acc.at[pl.ds(rbase, RPS)],
                    out_hbm.at[pl.ds(c * N_PAD + rbase, RPS)])


R = 1000


def _tc_in_body(x_ref, w_ref, cnt0_ref, cnt1_ref, g_ref, dis_ref):
    deg = 1.0 + cnt0_ref[0] + cnt1_ref[0]
    dis = lax.rsqrt(deg)
    dis_ref[...] = dis
    h = jnp.dot(x_ref[...], w_ref[...], preferred_element_type=jnp.float32)
    g_ref[0] = h * dis


def _tc_layer1(x, W1, cnt):
    return pl.pallas_call(
        _tc_in_body,
        grid=(N // R, NC),
        in_specs=[
            pl.BlockSpec((R, D_IN), lambda r, c: (r, 0)),
            pl.BlockSpec((D_IN, 128), lambda r, c: (0, c)),
            pl.BlockSpec((1, R, 1), lambda r, c: (0, r, 0)),
            pl.BlockSpec((1, R, 1), lambda r, c: (1, r, 0)),
        ],
        out_specs=[
            pl.BlockSpec((1, R, 128), lambda r, c: (c, r, 0)),
            pl.BlockSpec((R, 1), lambda r, c: (r, 0)),
        ],
        out_shape=[
            jax.ShapeDtypeStruct((NC, N_PAD, 128), jnp.float32),
            jax.ShapeDtypeStruct((N, 1), jnp.float32),
        ],
    )(x, W1, cnt, cnt)


def _tc_mid_body(a0_ref, a1_ref, dis_ref, b_ref, w_ref, h_ref, g_ref):
    dis = dis_ref[...]
    h = jnp.concatenate([a0_ref[0], a1_ref[0]], axis=1)
    h = jnp.maximum(h * dis + b_ref[...], 0.0)
    h_ref[...] = h
    g = jnp.dot(h, w_ref[...], preferred_element_type=jnp.float32)
    g_ref[0] = g * dis


def _tc_mid(a, dis, b, W):
    return pl.pallas_call(
        _tc_mid_body,
        grid=(N // R, NC),
        in_specs=[
            pl.BlockSpec((1, R, 128), lambda r, c: (0, r, 0)),
            pl.BlockSpec((1, R, 128), lambda r, c: (1, r, 0)),
            pl.BlockSpec((R, 1), lambda r, c: (r, 0)),
            pl.BlockSpec((1, HID), lambda r, c: (0, 0)),
            pl.BlockSpec((HID, 128), lambda r, c: (0, c)),
        ],
        out_specs=[
            pl.BlockSpec((R, HID), lambda r, c: (r, 0)),
            pl.BlockSpec((1, R, 128), lambda r, c: (c, r, 0)),
        ],
        out_shape=[
            jax.ShapeDtypeStruct((N, HID), jnp.float32),
            jax.ShapeDtypeStruct((NC, N_PAD, 128), jnp.float32),
        ],
    )(a, a, dis, b.reshape(1, HID), W)


def _tc_last_body(a0_ref, a1_ref, dis_ref, b_ref, w_ref, h_ref, g_ref):
    dis = dis_ref[...]
    h = jnp.concatenate([a0_ref[0], a1_ref[0]], axis=1)
    h = jnp.maximum(h * dis + b_ref[...], 0.0)
    h_ref[...] = h
    g = jnp.dot(h, w_ref[...], preferred_element_type=jnp.float32)
    g_ref[...] = g * dis


def _tc_last(a, dis, b, W):
    return pl.pallas_call(
        _tc_last_body,
        grid=(N // R,),
        in_specs=[
            pl.BlockSpec((1, R, 128), lambda r: (0, r, 0)),
            pl.BlockSpec((1, R, 128), lambda r: (1, r, 0)),
            pl.BlockSpec((R, 1), lambda r: (r, 0)),
            pl.BlockSpec((1, HID), lambda r: (0, 0)),
            pl.BlockSpec((HID, D_OUT), lambda r: (0, 0)),
        ],
        out_specs=[
            pl.BlockSpec((R, HID), lambda r: (r, 0)),
            pl.BlockSpec((R, D_OUT), lambda r: (r, 0)),
        ],
        out_shape=[
            jax.ShapeDtypeStruct((N, HID), jnp.float32),
            jax.ShapeDtypeStruct((N_PAD, D_OUT), jnp.float32),
        ],
    )(a, a, dis, b.reshape(1, HID), W)


def _tc_final_body(a0_ref, a1_ref, g_ref, dis_ref, b_ref, o_ref):
    o_ref[...] = (a0_ref[0] + a1_ref[0] - g_ref[...]) * dis_ref[...] + b_ref[...]


def _tc_final(a, g3, dis, b):
    return pl.pallas_call(
        _tc_final_body,
        grid=(N // R,),
        in_specs=[
            pl.BlockSpec((1, R, D_OUT), lambda r: (0, r, 0)),
            pl.BlockSpec((1, R, D_OUT), lambda r: (1, r, 0)),
            pl.BlockSpec((R, D_OUT), lambda r: (r, 0)),
            pl.BlockSpec((R, 1), lambda r: (r, 0)),
            pl.BlockSpec((1, D_OUT), lambda r: (0, 0)),
        ],
        out_specs=pl.BlockSpec((R, D_OUT), lambda r: (r, 0)),
        out_shape=jax.ShapeDtypeStruct((N, D_OUT), jnp.float32),
    )(a, a, g3, dis, b.reshape(1, D_OUT))


@jax.jit
def kernel(x, edge_index, W1, b1, W2, b2, W3, b3):
    npad = E_PAD - E
    src_p = jnp.concatenate([edge_index[0], jnp.zeros((npad,), jnp.int32)])
    pad_dst = DUMMY + jnp.arange(npad, dtype=jnp.int32) % (N_PAD - N)
    dst_p = jnp.concatenate([edge_index[1], pad_dst])
    src_both = jnp.concatenate([src_p, src_p + N_PAD]).reshape(2 * EROWS, CHUNK)
    src2 = src_p.reshape(EROWS, CHUNK)
    dst2 = dst_p.reshape(EROWS, CHUNK)

    cnt = _deg_kernel(dst2).reshape(NC, N_PAD, 1)
    g1, dis = _tc_layer1(x, W1, cnt)

    a1 = _prop_wide_kernel(g1.reshape(NC * N_PAD, 128), src_both, dst2)
    h1, g2 = _tc_mid(a1.reshape(NC, N_PAD, 128), dis, b1, W2)

    a2 = _prop_wide_kernel(g2.reshape(NC * N_PAD, 128), src_both, dst2)
    h2, g3 = _tc_last(a2.reshape(NC, N_PAD, 128), dis, b2, W3)

    a3 = _prop_narrow_kernel(g3, src2, dst2)
    out = _tc_final(a3.reshape(NC, N_PAD, 128), g3, dis, b3)
    return (out, h1, h2)


# speedup vs baseline: 1.6687x; 1.0009x over previous
import functools

import jax
import jax.numpy as jnp
from jax import lax
from jax.experimental import pallas as pl
from jax.experimental.pallas import tpu as pltpu
from jax.experimental.pallas import tpu_sc as plsc

N = 10000
E = 320000
D_IN = 128
HID = 256
D_OUT = 128

NC = 2
NS = 16
LANES = 16

CHUNK = 128
NBUF = 2
DUMMY = N
N_PAD = 10240
RPS = N_PAD // NS
E_PAD = 327680
CH_FULL = E_PAD // (NS * CHUNK)
CH_HALF = E_PAD // (NC * NS * CHUNK)
EROWS = E_PAD // CHUNK

_sc_mesh = plsc.VectorSubcoreMesh(
    core_axis_name="c", subcore_axis_name="s", num_cores=NC, num_subcores=NS)


@functools.partial(
    pl.kernel,
    out_type=jax.ShapeDtypeStruct((NC * N_PAD,), jnp.float32),
    mesh=_sc_mesh,
    compiler_params=pltpu.CompilerParams(needs_layout_passes=False),
    scratch_types=[
        pltpu.VMEM((CH_HALF, CHUNK), jnp.int32),
        pltpu.VMEM((N_PAD,), jnp.float32),
        pltpu.VMEM((NS, RPS), jnp.float32),
        pltpu.VMEM_SHARED((NS, N_PAD), jnp.float32),
    ],
)
def _deg_kernel(dst_hbm, cnt_hbm, didx, cnt_v, red_v, stage_s):
    c = lax.axis_index("c")
    s = lax.axis_index("s")
    zero16 = jnp.zeros((LANES,), jnp.float32)
    one16 = jnp.ones((LANES,), jnp.float32)

    row0 = (c * NS + s) * CH_HALF
    pltpu.sync_copy(dst_hbm.at[pl.ds(row0, CH_HALF)], didx)

    @pl.loop(0, N_PAD // LANES)
    def _zero(i):
        cnt_v[pl.ds(i * LANES, LANES)] = zero16

    @pl.loop(0, CH_HALF)
    def _count(i):
        for j in range(CHUNK // LANES):
            idx = didx[i, pl.ds(j * LANES, LANES)]
            plsc.addupdate_scatter(cnt_v, [idx], one16)

    pltpu.sync_copy(cnt_v, stage_s.at[s])
    plsc.subcore_barrier()

    rbase = s * RPS
    pltpu.sync_copy(stage_s.at[:, pl.ds(rbase, RPS)], red_v)

    @pl.loop(0, RPS // LANES)
    def _reduce(i):
        sl = pl.ds(i * LANES, LANES)
        acc16 = red_v[0, sl]
        for t in range(1, NS):
            acc16 = acc16 + red_v[t, sl]
        cnt_v[sl] = acc16

    pltpu.sync_copy(cnt_v.at[pl.ds(0, RPS)],
                    cnt_hbm.at[pl.ds(c * N_PAD + rbase, RPS)])


GRP = 8


def _edge_ring(g_hbm, src2d, dst2d, srow, drow, nworkers, acc,
               sidx, didx, rows, sem_i, sem_g, sem_s, n_chunks):
    def sgrow(G):
        return srow + 8 * nworkers * G

    def dgrow(G):
        return drow + 8 * nworkers * G

    def load_group(G, slot):
        pltpu.async_copy(src2d.at[pl.ds(sgrow(G), GRP)],
                         sidx.at[pl.ds(slot * GRP, GRP)], sem_i.at[slot])
        pltpu.async_copy(dst2d.at[pl.ds(dgrow(G), GRP)],
                         didx.at[pl.ds(slot * GRP, GRP)], sem_i.at[slot])

    def wait_group(slot):
        pltpu.make_async_copy(src2d.at[pl.ds(srow, GRP)],
                              sidx.at[pl.ds(slot * GRP, GRP)],
                              sem_i.at[slot]).wait()
        pltpu.make_async_copy(dst2d.at[pl.ds(drow, GRP)],
                              didx.at[pl.ds(slot * GRP, GRP)],
                              sem_i.at[slot]).wait()

    def wait_s(b):
        pltpu.make_async_copy(rows[b], acc.at[pl.ds(0, CHUNK)],
                              sem_s.at[b]).wait()

    n_groups = n_chunks // GRP
    load_group(0, 0)
    wait_group(0)
    pltpu.async_copy(g_hbm.at[sidx.at[0]], rows[0], sem_g.at[0])

    @pl.loop(0, n_groups // 2)
    def _pair(grp):
        for bG in range(2):
            for k in range(GRP):
                i = (grp * 2 + bG) * GRP + k
                b = k % NBUF
                o = 1 - b

                @pl.when(i > 0)
                def _():
                    wait_s(o)

                if k == 0:
                    @pl.when(grp * 2 + bG + 1 < n_groups)
                    def _():
                        load_group(grp * 2 + bG + 1, 1 - bG)

                pltpu.make_async_copy(
                    g_hbm.at[pl.ds(0, CHUNK)], rows[b], sem_g.at[b]).wait()
                pltpu.async_copy(rows[b], acc.at[didx.at[bG * GRP + k]],
                                 sem_s.at[b], add=True)

                if k < GRP - 1:
                    @pl.when(i + 1 < n_chunks)
                    def _():
                        pltpu.async_copy(g_hbm.at[sidx.at[bG * GRP + k + 1]],
                                         rows[o], sem_g.at[o])
                else:
                    @pl.when(i + 1 < n_chunks)
                    def _():
                        wait_group(1 - bG)
                        pltpu.async_copy(g_hbm.at[sidx.at[(1 - bG) * GRP]],
                                         rows[o], sem_g.at[o])

    wait_s((n_chunks - 1) % NBUF)


@functools.partial(
    pl.kernel,
    out_type=jax.ShapeDtypeStruct((NC * N_PAD, 128), jnp.float32),
    mesh=_sc_mesh,
    scratch_types=[
        pltpu.VMEM((2 * GRP, CHUNK), jnp.int32),
        pltpu.VMEM((2 * GRP, CHUNK), jnp.int32),
        [pltpu.VMEM((CHUNK, 128), jnp.float32)] * NBUF,
        pltpu.VMEM_SHARED((N_PAD, 128), jnp.float32),
        pltpu.SemaphoreType.DMA((2,)),
        pltpu.SemaphoreType.DMA((NBUF,)),
        pltpu.SemaphoreType.DMA((NBUF,)),
    ],
)
def _prop_wide_kernel(g_hbm, srcb_hbm, dst_hbm, out_hbm,
                      sidx, didx, rows, acc, sem_i, sem_g, sem_s):
    c = lax.axis_index("c")
    s = lax.axis_index("s")
    rbase = s * RPS
    pltpu.sync_copy(g_hbm.at[pl.ds(c * N_PAD + rbase, RPS)],
                    acc.at[pl.ds(rbase, RPS)])
    plsc.subcore_barrier()

    _edge_ring(g_hbm, srcb_hbm, dst_hbm, c * EROWS + 8 * s, 8 * s, NS,
               acc, sidx, didx, rows, sem_i, sem_g, sem_s, CH_FULL)

    plsc.subcore_barrier()
    pltpu.sync_copy(acc.at[pl.ds(rbase, RPS)],
                    out_hbm.at[pl.ds(c * N_PAD + rbase, RPS)])


@functools.partial(
    pl.kernel,
    out_type=jax.ShapeDtypeStruct((NC * N_PAD, 128), jnp.float32),
    mesh=_sc_mesh,
    scratch_types=[
        pltpu.VMEM((2 * GRP, CHUNK), jnp.int32),
        pltpu.VMEM((2 * GRP, CHUNK), jnp.int32),
        [pltpu.VMEM((CHUNK, 128), jnp.float32)] * NBUF,
        pltpu.VMEM_SHARED((N_PAD, 128), jnp.float32),
        pltpu.SemaphoreType.DMA((2,)),
        pltpu.SemaphoreType.DMA((NBUF,)),
        pltpu.SemaphoreType.DMA((NBUF,)),
    ],
)
def _prop_narrow_kernel(g_hbm, src_hbm, dst_hbm, out_hbm,
                        sidx, didx, rows, acc, sem_i, sem_g, sem_s):
    c = lax.axis_index("c")
    s = lax.axis_index("s")
    rbase = s * RPS
    w = c * NS + s
    pltpu.sync_copy(g_hbm.at[pl.ds(rbase, RPS)], acc.at[pl.ds(rbase, RPS)])
    plsc.subcore_barrier()

    _edge_ring(g_hbm, src_hbm, dst_hbm, 8 * w, 8 * w, NC * NS, acc,
               sidx, didx, rows, sem_i, sem_g, sem_s, CH_HALF)

    plsc.subcore_barrier()
    pltpu.sync_copy(acc.at[pl.ds(rbase, RPS)],
                    out_hbm.at[pl.ds(c * N_PAD + rbase, RPS)])


R = 1000


def _tc_in_body(x_ref, w_ref, cnt0_ref, cnt1_ref, g_ref, dis_ref):
    deg = 1.0 + cnt0_ref[0] + cnt1_ref[0]
    dis = lax.rsqrt(deg)
    dis_ref[...] = dis
    h = jnp.dot(x_ref[...], w_ref[...], preferred_element_type=jnp.float32)
    g_ref[0] = h * dis


def _tc_layer1(x, W1, cnt):
    return pl.pallas_call(
        _tc_in_body,
        grid=(N // R, NC),
        in_specs=[
            pl.BlockSpec((R, D_IN), lambda r, c: (r, 0)),
            pl.BlockSpec((D_IN, 128), lambda r, c: (0, c)),
            pl.BlockSpec((1, R, 1), lambda r, c: (0, r, 0)),
            pl.BlockSpec((1, R, 1), lambda r, c: (1, r, 0)),
        ],
        out_specs=[
            pl.BlockSpec((1, R, 128), lambda r, c: (c, r, 0)),
            pl.BlockSpec((R, 1), lambda r, c: (r, 0)),
        ],
        out_shape=[
            jax.ShapeDtypeStruct((NC, N_PAD, 128), jnp.float32),
            jax.ShapeDtypeStruct((N, 1), jnp.float32),
        ],
    )(x, W1, cnt, cnt)


def _tc_mid_body(a0_ref, a1_ref, dis_ref, b_ref, w_ref, h_ref, g_ref):
    dis = dis_ref[...]
    h = jnp.concatenate([a0_ref[0], a1_ref[0]], axis=1)
    h = jnp.maximum(h * dis + b_ref[...], 0.0)
    h_ref[...] = h
    g = jnp.dot(h, w_ref[...], preferred_element_type=jnp.float32)
    g_ref[0] = g * dis


def _tc_mid(a, dis, b, W):
    return pl.pallas_call(
        _tc_mid_body,
        grid=(N // R, NC),
        in_specs=[
            pl.BlockSpec((1, R, 128), lambda r, c: (0, r, 0)),
            pl.BlockSpec((1, R, 128), lambda r, c: (1, r, 0)),
            pl.BlockSpec((R, 1), lambda r, c: (r, 0)),
            pl.BlockSpec((1, HID), lambda r, c: (0, 0)),
            pl.BlockSpec((HID, 128), lambda r, c: (0, c)),
        ],
        out_specs=[
            pl.BlockSpec((R, HID), lambda r, c: (r, 0)),
            pl.BlockSpec((1, R, 128), lambda r, c: (c, r, 0)),
        ],
        out_shape=[
            jax.ShapeDtypeStruct((N, HID), jnp.float32),
            jax.ShapeDtypeStruct((NC, N_PAD, 128), jnp.float32),
        ],
    )(a, a, dis, b.reshape(1, HID), W)


def _tc_last_body(a0_ref, a1_ref, dis_ref, b_ref, w_ref, h_ref, g_ref):
    dis = dis_ref[...]
    h = jnp.concatenate([a0_ref[0], a1_ref[0]], axis=1)
    h = jnp.maximum(h * dis + b_ref[...], 0.0)
    h_ref[...] = h
    g = jnp.dot(h, w_ref[...], preferred_element_type=jnp.float32)
    g_ref[...] = g * dis


def _tc_last(a, dis, b, W):
    return pl.pallas_call(
        _tc_last_body,
        grid=(N // R,),
        in_specs=[
            pl.BlockSpec((1, R, 128), lambda r: (0, r, 0)),
            pl.BlockSpec((1, R, 128), lambda r: (1, r, 0)),
            pl.BlockSpec((R, 1), lambda r: (r, 0)),
            pl.BlockSpec((1, HID), lambda r: (0, 0)),
            pl.BlockSpec((HID, D_OUT), lambda r: (0, 0)),
        ],
        out_specs=[
            pl.BlockSpec((R, HID), lambda r: (r, 0)),
            pl.BlockSpec((R, D_OUT), lambda r: (r, 0)),
        ],
        out_shape=[
            jax.ShapeDtypeStruct((N, HID), jnp.float32),
            jax.ShapeDtypeStruct((N_PAD, D_OUT), jnp.float32),
        ],
    )(a, a, dis, b.reshape(1, HID), W)


def _tc_final_body(a0_ref, a1_ref, g_ref, dis_ref, b_ref, o_ref):
    o_ref[...] = (a0_ref[0] + a1_ref[0] - g_ref[...]) * dis_ref[...] + b_ref[...]


def _tc_final(a, g3, dis, b):
    return pl.pallas_call(
        _tc_final_body,
        grid=(N // R,),
        in_specs=[
            pl.BlockSpec((1, R, D_OUT), lambda r: (0, r, 0)),
            pl.BlockSpec((1, R, D_OUT), lambda r: (1, r, 0)),
            pl.BlockSpec((R, D_OUT), lambda r: (r, 0)),
            pl.BlockSpec((R, 1), lambda r: (r, 0)),
            pl.BlockSpec((1, D_OUT), lambda r: (0, 0)),
        ],
        out_specs=pl.BlockSpec((R, D_OUT), lambda r: (r, 0)),
        out_shape=jax.ShapeDtypeStruct((N, D_OUT), jnp.float32),
    )(a, a, g3, dis, b.reshape(1, D_OUT))


@jax.jit
def kernel(x, edge_index, W1, b1, W2, b2, W3, b3):
    npad = E_PAD - E
    src_p = jnp.concatenate([edge_index[0], jnp.zeros((npad,), jnp.int32)])
    pad_dst = DUMMY + jnp.arange(npad, dtype=jnp.int32) % (N_PAD - N)
    dst_p = jnp.concatenate([edge_index[1], pad_dst])
    src_both = jnp.concatenate([src_p, src_p + N_PAD]).reshape(2 * EROWS, CHUNK)
    src2 = src_p.reshape(EROWS, CHUNK)
    dst2 = dst_p.reshape(EROWS, CHUNK)

    cnt = _deg_kernel(dst2).reshape(NC, N_PAD, 1)
    g1, dis = _tc_layer1(x, W1, cnt)

    a1 = _prop_wide_kernel(g1.reshape(NC * N_PAD, 128), src_both, dst2)
    h1, g2 = _tc_mid(a1.reshape(NC, N_PAD, 128), dis, b1, W2)

    a2 = _prop_wide_kernel(g2.reshape(NC * N_PAD, 128), src_both, dst2)
    h2, g3 = _tc_last(a2.reshape(NC, N_PAD, 128), dis, b2, W3)

    a3 = _prop_narrow_kernel(g3, src2, dst2)
    out = _tc_final(a3.reshape(NC, N_PAD, 128), g3, dis, b3)
    return (out, h1, h2)


# speedup vs baseline: 3.5406x; 2.1218x over previous
import functools

import jax
import jax.numpy as jnp
from jax import lax
from jax.experimental import pallas as pl
from jax.experimental.pallas import tpu as pltpu
from jax.experimental.pallas import tpu_sc as plsc

N = 10000
E = 320000
D_IN = 128
HID = 256
D_OUT = 128

NC = 2
NS = 16
LANES = 16

CHUNK = 128
NBUF = 2
DUMMY = N
N_PAD = 10240
RPS = N_PAD // NS
E_PAD = 327680
CH_FULL = E_PAD // (NS * CHUNK)
CH_HALF = E_PAD // (NC * NS * CHUNK)
EROWS = E_PAD // CHUNK

_sc_mesh = plsc.VectorSubcoreMesh(
    core_axis_name="c", subcore_axis_name="s", num_cores=NC, num_subcores=NS)


@functools.partial(
    pl.kernel,
    out_type=jax.ShapeDtypeStruct((NC * N_PAD,), jnp.float32),
    mesh=_sc_mesh,
    compiler_params=pltpu.CompilerParams(needs_layout_passes=False),
    scratch_types=[
        pltpu.VMEM((CH_HALF, CHUNK), jnp.int32),
        pltpu.VMEM((N_PAD,), jnp.float32),
        pltpu.VMEM((NS, RPS), jnp.float32),
        pltpu.VMEM_SHARED((NS, N_PAD), jnp.float32),
    ],
)
def _deg_kernel(dst_hbm, cnt_hbm, didx, cnt_v, red_v, stage_s):
    c = lax.axis_index("c")
    s = lax.axis_index("s")
    zero16 = jnp.zeros((LANES,), jnp.float32)
    one16 = jnp.ones((LANES,), jnp.float32)

    row0 = (c * NS + s) * CH_HALF
    pltpu.sync_copy(dst_hbm.at[pl.ds(row0, CH_HALF)], didx)

    @pl.loop(0, N_PAD // LANES)
    def _zero(i):
        cnt_v[pl.ds(i * LANES, LANES)] = zero16

    @pl.loop(0, CH_HALF)
    def _count(i):
        for j in range(CHUNK // LANES):
            idx = didx[i, pl.ds(j * LANES, LANES)]
            plsc.addupdate_scatter(cnt_v, [idx], one16)

    pltpu.sync_copy(cnt_v, stage_s.at[s])
    plsc.subcore_barrier()

    rbase = s * RPS
    pltpu.sync_copy(stage_s.at[:, pl.ds(rbase, RPS)], red_v)

    @pl.loop(0, RPS // LANES)
    def _reduce(i):
        sl = pl.ds(i * LANES, LANES)
        acc16 = red_v[0, sl]
        for t in range(1, NS):
            acc16 = acc16 + red_v[t, sl]
        cnt_v[sl] = acc16

    pltpu.sync_copy(cnt_v.at[pl.ds(0, RPS)],
                    cnt_hbm.at[pl.ds(c * N_PAD + rbase, RPS)])


GRP = 8


def _edge_ring(g_hbm, src2d, dst2d, srow, drow, nworkers, acc,
               sidx, didx, rows, sem_i, sem_g, sem_s, n_chunks):
    def sgrow(G):
        return srow + 8 * nworkers * G

    def dgrow(G):
        return drow + 8 * nworkers * G

    def load_group(G, slot):
        pltpu.async_copy(src2d.at[pl.ds(sgrow(G), GRP)],
                         sidx.at[pl.ds(slot * GRP, GRP)], sem_i.at[slot])
        pltpu.async_copy(dst2d.at[pl.ds(dgrow(G), GRP)],
                         didx.at[pl.ds(slot * GRP, GRP)], sem_i.at[slot])

    def wait_group(slot):
        pltpu.make_async_copy(src2d.at[pl.ds(srow, GRP)],
                              sidx.at[pl.ds(slot * GRP, GRP)],
                              sem_i.at[slot]).wait()
        pltpu.make_async_copy(dst2d.at[pl.ds(drow, GRP)],
                              didx.at[pl.ds(slot * GRP, GRP)],
                              sem_i.at[slot]).wait()

    def wait_s(b):
        pltpu.make_async_copy(rows[b], acc.at[pl.ds(0, CHUNK)],
                              sem_s.at[b]).wait()

    n_groups = n_chunks // GRP
    load_group(0, 0)
    wait_group(0)
    pltpu.async_copy(g_hbm.at[sidx.at[0]], rows[0], sem_g.at[0])

    @pl.loop(0, n_groups // 2)
    def _pair(grp):
        for bG in range(2):
            for k in range(GRP):
                i = (grp * 2 + bG) * GRP + k
                b = k % NBUF
                o = 1 - b

                @pl.when(i > 0)
                def _():
                    wait_s(o)

                if k == 0:
                    @pl.when(grp * 2 + bG + 1 < n_groups)
                    def _():
                        load_group(grp * 2 + bG + 1, 1 - bG)

                pltpu.make_async_copy(
                    g_hbm.at[pl.ds(0, CHUNK)], rows[b], sem_g.at[b]).wait()
                pltpu.async_copy(rows[b], acc.at[didx.at[bG * GRP + k]],
                                 sem_s.at[b], add=True)

                if k < GRP - 1:
                    @pl.when(i + 1 < n_chunks)
                    def _():
                        pltpu.async_copy(g_hbm.at[sidx.at[bG * GRP + k + 1]],
                                         rows[o], sem_g.at[o])
                else:
                    @pl.when(i + 1 < n_chunks)
                    def _():
                        wait_group(1 - bG)
                        pltpu.async_copy(g_hbm.at[sidx.at[(1 - bG) * GRP]],
                                         rows[o], sem_g.at[o])

    wait_s((n_chunks - 1) % NBUF)


@functools.partial(
    pl.kernel,
    out_type=jax.ShapeDtypeStruct((NC * N_PAD, 128), jnp.float32),
    mesh=_sc_mesh,
    scratch_types=[
        pltpu.VMEM((2 * GRP, CHUNK), jnp.int32),
        pltpu.VMEM((2 * GRP, CHUNK), jnp.int32),
        [pltpu.VMEM((CHUNK, 128), jnp.float32)] * NBUF,
        pltpu.VMEM_SHARED((N_PAD, 128), jnp.float32),
        pltpu.SemaphoreType.DMA((2,)),
        pltpu.SemaphoreType.DMA((NBUF,)),
        pltpu.SemaphoreType.DMA((NBUF,)),
    ],
)
def _prop_wide_kernel(g_hbm, srcb_hbm, dst_hbm, out_hbm,
                      sidx, didx, rows, acc, sem_i, sem_g, sem_s):
    c = lax.axis_index("c")
    s = lax.axis_index("s")
    rbase = s * RPS
    pltpu.sync_copy(g_hbm.at[pl.ds(c * N_PAD + rbase, RPS)],
                    acc.at[pl.ds(rbase, RPS)])
    plsc.subcore_barrier()

    _edge_ring(g_hbm, srcb_hbm, dst_hbm, c * EROWS + 8 * s, 8 * s, NS,
               acc, sidx, didx, rows, sem_i, sem_g, sem_s, CH_FULL)

    plsc.subcore_barrier()
    pltpu.sync_copy(acc.at[pl.ds(rbase, RPS)],
                    out_hbm.at[pl.ds(c * N_PAD + rbase, RPS)])


@functools.partial(
    pl.kernel,
    out_type=jax.ShapeDtypeStruct((NC * N_PAD, 128), jnp.float32),
    mesh=_sc_mesh,
    scratch_types=[
        pltpu.VMEM((2 * GRP, CHUNK), jnp.int32),
        pltpu.VMEM((2 * GRP, CHUNK), jnp.int32),
        [pltpu.VMEM((CHUNK, 128), jnp.float32)] * NBUF,
        pltpu.VMEM_SHARED((N_PAD, 128), jnp.float32),
        pltpu.SemaphoreType.DMA((2,)),
        pltpu.SemaphoreType.DMA((NBUF,)),
        pltpu.SemaphoreType.DMA((NBUF,)),
    ],
)
def _prop_narrow_kernel(g_hbm, src_hbm, dst_hbm, out_hbm,
                        sidx, didx, rows, acc, sem_i, sem_g, sem_s):
    c = lax.axis_index("c")
    s = lax.axis_index("s")
    rbase = s * RPS
    w = c * NS + s
    pltpu.sync_copy(g_hbm.at[pl.ds(rbase, RPS)], acc.at[pl.ds(rbase, RPS)])
    plsc.subcore_barrier()

    _edge_ring(g_hbm, src_hbm, dst_hbm, 8 * w, 8 * w, NC * NS, acc,
               sidx, didx, rows, sem_i, sem_g, sem_s, CH_HALF)

    plsc.subcore_barrier()
    pltpu.sync_copy(acc.at[pl.ds(rbase, RPS)],
                    out_hbm.at[pl.ds(c * N_PAD + rbase, RPS)])


R = 1000


def _tc_in_body(x_ref, w_ref, cnt0_ref, cnt1_ref, g_ref, dis_ref):
    deg = 1.0 + cnt0_ref[0] + cnt1_ref[0]
    dis = lax.rsqrt(deg)
    dis_ref[...] = dis
    h = jnp.dot(x_ref[...], w_ref[...], preferred_element_type=jnp.float32)
    g_ref[0] = h * dis


def _tc_layer1(x, W1, cnt):
    return pl.pallas_call(
        _tc_in_body,
        grid=(N // R, NC),
        in_specs=[
            pl.BlockSpec((R, D_IN), lambda r, c: (r, 0)),
            pl.BlockSpec((D_IN, 128), lambda r, c: (0, c)),
            pl.BlockSpec((1, R, 1), lambda r, c: (0, r, 0)),
            pl.BlockSpec((1, R, 1), lambda r, c: (1, r, 0)),
        ],
        out_specs=[
            pl.BlockSpec((1, R, 128), lambda r, c: (c, r, 0)),
            pl.BlockSpec((R, 1), lambda r, c: (r, 0)),
        ],
        out_shape=[
            jax.ShapeDtypeStruct((NC, N_PAD, 128), jnp.float32),
            jax.ShapeDtypeStruct((N, 1), jnp.float32),
        ],
    )(x, W1, cnt, cnt)


def _tc_mid_body(a0_ref, a1_ref, dis_ref, b_ref, w_ref, h_ref, g_ref):
    dis = dis_ref[...]
    h = jnp.concatenate([a0_ref[0], a1_ref[0]], axis=1)
    h = jnp.maximum(h * dis + b_ref[...], 0.0)
    h_ref[...] = h
    g = jnp.dot(h, w_ref[...], preferred_element_type=jnp.float32)
    g_ref[0] = g * dis


def _tc_mid(a, dis, b, W):
    return pl.pallas_call(
        _tc_mid_body,
        grid=(N // R, NC),
        in_specs=[
            pl.BlockSpec((1, R, 128), lambda r, c: (0, r, 0)),
            pl.BlockSpec((1, R, 128), lambda r, c: (1, r, 0)),
            pl.BlockSpec((R, 1), lambda r, c: (r, 0)),
            pl.BlockSpec((1, HID), lambda r, c: (0, 0)),
            pl.BlockSpec((HID, 128), lambda r, c: (0, c)),
        ],
        out_specs=[
            pl.BlockSpec((R, HID), lambda r, c: (r, 0)),
            pl.BlockSpec((1, R, 128), lambda r, c: (c, r, 0)),
        ],
        out_shape=[
            jax.ShapeDtypeStruct((N, HID), jnp.float32),
            jax.ShapeDtypeStruct((NC, N_PAD, 128), jnp.float32),
        ],
    )(a, a, dis, b.reshape(1, HID), W)


def _tc_last_body(a0_ref, a1_ref, dis_ref, b_ref, w_ref, h_ref, g_ref):
    dis = dis_ref[...]
    h = jnp.concatenate([a0_ref[0], a1_ref[0]], axis=1)
    h = jnp.maximum(h * dis + b_ref[...], 0.0)
    h_ref[...] = h
    g = jnp.dot(h, w_ref[...], preferred_element_type=jnp.float32)
    g_ref[...] = g * dis


def _tc_last(a, dis, b, W):
    return pl.pallas_call(
        _tc_last_body,
        grid=(N // R,),
        in_specs=[
            pl.BlockSpec((1, R, 128), lambda r: (0, r, 0)),
            pl.BlockSpec((1, R, 128), lambda r: (1, r, 0)),
            pl.BlockSpec((R, 1), lambda r: (r, 0)),
            pl.BlockSpec((1, HID), lambda r: (0, 0)),
            pl.BlockSpec((HID, D_OUT), lambda r: (0, 0)),
        ],
        out_specs=[
            pl.BlockSpec((R, HID), lambda r: (r, 0)),
            pl.BlockSpec((R, D_OUT), lambda r: (r, 0)),
        ],
        out_shape=[
            jax.ShapeDtypeStruct((N, HID), jnp.float32),
            jax.ShapeDtypeStruct((N_PAD, D_OUT), jnp.float32),
        ],
    )(a, a, dis, b.reshape(1, HID), W)


def _tc_final_body(a0_ref, a1_ref, g_ref, dis_ref, b_ref, o_ref):
    o_ref[...] = (a0_ref[0] + a1_ref[0] - g_ref[...]) * dis_ref[...] + b_ref[...]


def _tc_final(a, g3, dis, b):
    return pl.pallas_call(
        _tc_final_body,
        grid=(N // R,),
        in_specs=[
            pl.BlockSpec((1, R, D_OUT), lambda r: (0, r, 0)),
            pl.BlockSpec((1, R, D_OUT), lambda r: (1, r, 0)),
            pl.BlockSpec((R, D_OUT), lambda r: (r, 0)),
            pl.BlockSpec((R, 1), lambda r: (r, 0)),
            pl.BlockSpec((1, D_OUT), lambda r: (0, 0)),
        ],
        out_specs=pl.BlockSpec((R, D_OUT), lambda r: (r, 0)),
        out_shape=jax.ShapeDtypeStruct((N, D_OUT), jnp.float32),
    )(a, a, g3, dis, b.reshape(1, D_OUT))


@jax.jit
def kernel(x, edge_index, W1, b1, W2, b2, W3, b3):
    npad = E_PAD - E
    pad_iota = jnp.arange(npad, dtype=jnp.int32)
    src_p = jnp.concatenate([edge_index[0], pad_iota % N])
    pad_dst = DUMMY + pad_iota % (N_PAD - N)
    dst_p = jnp.concatenate([edge_index[1], pad_dst])
    src_both = jnp.concatenate([src_p, src_p + N_PAD]).reshape(2 * EROWS, CHUNK)
    src2 = src_p.reshape(EROWS, CHUNK)
    dst2 = dst_p.reshape(EROWS, CHUNK)

    cnt = _deg_kernel(dst2).reshape(NC, N_PAD, 1)
    g1, dis = _tc_layer1(x, W1, cnt)

    a1 = _prop_wide_kernel(g1.reshape(NC * N_PAD, 128), src_both, dst2)
    h1, g2 = _tc_mid(a1.reshape(NC, N_PAD, 128), dis, b1, W2)

    a2 = _prop_wide_kernel(g2.reshape(NC * N_PAD, 128), src_both, dst2)
    h2, g3 = _tc_last(a2.reshape(NC, N_PAD, 128), dis, b2, W3)

    a3 = _prop_narrow_kernel(g3, src2, dst2)
    out = _tc_final(a3.reshape(NC, N_PAD, 128), g3, dis, b3)
    return (out, h1, h2)
